# Initial kernel scaffold; baseline (speedup 1.0000x reference)
import jax
import jax.numpy as jnp
from jax.experimental import pallas as pl


def kernel(x, indices):
    raise NotImplementedError("write your pallas kernel here")


# speedup vs baseline: 66.7667x; 66.7667x over previous
import jax
import jax.numpy as jnp
from jax import lax
from jax.experimental import pallas as pl
from jax.experimental.pallas import tpu as pltpu
from jax.experimental.pallas import tpu_sc as plsc

_NW = 32


def _make_unpool(nplane, nin, nout):
    planes_per_w = nplane // _NW
    zero_iters = nout // (16 * 8)
    scat_iters = nin // (16 * 4)

    def body(x_hbm, idx_hbm, out_hbm, x_v, idx_v, out_v, sem):
        wid = lax.axis_index("s") * 2 + lax.axis_index("c")
        zero16 = jnp.zeros((16,), jnp.float32)

        def plane_body(p, carry):
            plane = wid * planes_per_w + p
            cpx = pltpu.async_copy(x_hbm.at[plane], x_v, sem)
            cpi = pltpu.async_copy(idx_hbm.at[plane], idx_v, sem)

            def zbody(i, c):
                base = i * 128
                for u in range(8):
                    out_v[pl.ds(base + u * 16, 16)] = zero16
                return c

            lax.fori_loop(0, zero_iters, zbody, 0)
            cpx.wait()
            cpi.wait()

            def sbody(j, c):
                base = j * 64
                for u in range(4):
                    off = base + u * 16
                    idx = idx_v[pl.ds(off, 16)]
                    vals = x_v[pl.ds(off, 16)]
                    plsc.store_scatter(out_v, [idx], vals)
                return c

            lax.fori_loop(0, scat_iters, sbody, 0)
            pltpu.async_copy(out_v, out_hbm.at[plane], sem).wait()
            return carry

        lax.fori_loop(0, planes_per_w, plane_body, 0)

    mesh = plsc.VectorSubcoreMesh(core_axis_name="c", subcore_axis_name="s")
    return pl.kernel(
        body,
        mesh=mesh,
        compiler_params=pltpu.CompilerParams(needs_layout_passes=False),
        out_type=jax.ShapeDtypeStruct((nplane, nout), jnp.float32),
        scratch_types=[
            pltpu.VMEM((nin,), jnp.float32),
            pltpu.VMEM((nin,), jnp.int32),
            pltpu.VMEM((nout,), jnp.float32),
            pltpu.SemaphoreType.DMA,
        ],
    )


def kernel(x, indices):
    B, C, H, W = x.shape
    nplane = B * C
    nin = H * W
    nout = 4 * H * W
    xf = x.reshape(nplane, nin)
    idxf = indices.astype(jnp.int32).reshape(nplane, nin)
    out = _make_unpool(nplane, nin, nout)(xf, idxf)
    return out.reshape(B, C, 2 * H, 2 * W)


# speedup vs baseline: 69.5241x; 1.0413x over previous
import jax
import jax.numpy as jnp
from jax import lax
from jax.experimental import pallas as pl
from jax.experimental.pallas import tpu as pltpu
from jax.experimental.pallas import tpu_sc as plsc

_NW = 32


def _make_unpool(nplane, nin, nout):
    planes_per_w = nplane // _NW
    zero_iters = nout // (16 * 16)
    scat_iters = nin // (16 * 8)

    def body(x_hbm, idx_hbm, out_hbm, x_v, idx_v, out0, out1, sem_in, sem_out):
        wid = lax.axis_index("s") * 2 + lax.axis_index("c")
        base_plane = wid * planes_per_w
        zero16 = jnp.zeros((16,), jnp.float32)
        outs = [out0, out1]

        def zero_fill(ob):
            def zbody(i, c):
                base = i * 256
                for u in range(16):
                    ob[pl.ds(base + u * 16, 16)] = zero16
                return c

            lax.fori_loop(0, zero_iters, zbody, 0)

        def scatter(ob):
            def sbody(j, c):
                base = j * 128
                for u in range(8):
                    off = base + u * 16
                    plsc.store_scatter(ob, [idx_v[pl.ds(off, 16)]],
                                       x_v[pl.ds(off, 16)])
                return c

            lax.fori_loop(0, scat_iters, sbody, 0)

        cin = [
            pltpu.async_copy(x_hbm.at[base_plane], x_v, sem_in),
            pltpu.async_copy(idx_hbm.at[base_plane], idx_v, sem_in),
        ]
        zero_fill(out0)
        zero_fill(out1)
        pending = [None, None]
        for p in range(planes_per_w):
            ob = outs[p % 2]
            if pending[p % 2] is not None:
                pending[p % 2].wait()
                zero_fill(ob)
            cin[0].wait()
            cin[1].wait()
            scatter(ob)
            pending[p % 2] = pltpu.async_copy(
                ob, out_hbm.at[base_plane + p], sem_out)
            if p + 1 < planes_per_w:
                cin = [
                    pltpu.async_copy(x_hbm.at[base_plane + p + 1], x_v, sem_in),
                    pltpu.async_copy(idx_hbm.at[base_plane + p + 1], idx_v,
                                     sem_in),
                ]
        pending[0].wait()
        pending[1].wait()

    mesh = plsc.VectorSubcoreMesh(core_axis_name="c", subcore_axis_name="s")
    return pl.kernel(
        body,
        mesh=mesh,
        compiler_params=pltpu.CompilerParams(needs_layout_passes=False),
        out_type=jax.ShapeDtypeStruct((nplane, nout), jnp.float32),
        scratch_types=[
            pltpu.VMEM((nin,), jnp.float32),
            pltpu.VMEM((nin,), jnp.int32),
            pltpu.VMEM((nout,), jnp.float32),
            pltpu.VMEM((nout,), jnp.float32),
            pltpu.SemaphoreType.DMA,
            pltpu.SemaphoreType.DMA,
        ],
    )


def kernel(x, indices):
    B, C, H, W = x.shape
    nplane = B * C
    nin = H * W
    nout = 4 * H * W
    xf = x.reshape(nplane, nin)
    idxf = indices.astype(jnp.int32).reshape(nplane, nin)
    out = _make_unpool(nplane, nin, nout)(xf, idxf)
    return out.reshape(B, C, 2 * H, 2 * W)
